# baseline (speedup 1.0000x reference)
import jax
import jax.numpy as jnp
from jax.experimental import pallas as pl
from jax.experimental.pallas import tpu as pltpu

_BM = 200
_NBUF = 4


def _mm_manual(adj_hbm, emb_ref, out_ref, buf, sems):
    nsteps = pl.num_programs(0)
    i = pl.program_id(0)

    def _copy(step, slot):
        return pltpu.make_async_copy(
            adj_hbm.at[pl.ds(step * _BM, _BM), :], buf.at[slot], sems.at[slot])

    @pl.when(i == 0)
    def _warmup():
        for b in range(_NBUF):
            _copy(b, b).start()

    slot = jax.lax.rem(i, _NBUF)
    _copy(i, slot).wait()
    out_ref[...] = jax.lax.dot_general(
        buf[slot], emb_ref[...],
        dimension_numbers=(((1,), (0,)), ((), ())),
        precision=jax.lax.Precision.DEFAULT,
        preferred_element_type=jnp.float32)

    nxt = i + _NBUF
    @pl.when(nxt < nsteps)
    def _prefetch():
        _copy(nxt, slot).start()


def kernel(adj, embeds):
    m, k = adj.shape
    n = embeds.shape[1]
    return pl.pallas_call(
        _mm_manual,
        grid=(m // _BM,),
        in_specs=[
            pl.BlockSpec(memory_space=pltpu.MemorySpace.HBM),
            pl.BlockSpec((k, n), lambda i: (0, 0)),
        ],
        out_specs=pl.BlockSpec((_BM, n), lambda i: (i, 0)),
        out_shape=jax.ShapeDtypeStruct((m, n), jnp.float32),
        scratch_shapes=[
            pltpu.VMEM((_NBUF, _BM, k), jnp.float32),
            pltpu.SemaphoreType.DMA((_NBUF,)),
        ],
    )(adj, embeds)
